# baseline (speedup 1.0000x reference)
import functools

import jax
import jax.numpy as jnp
from jax import lax
from jax.experimental import pallas as pl
from jax.experimental.pallas import tpu as pltpu
from jax.experimental.pallas import tpu_sc as plsc

_RATIO = 0.1
_NW = 32
_NB = 2048
_NBC = _NB // 16


def _pat16(a, b):
    bits = jax.lax.bitcast_convert_type(jnp.abs(a - b), jnp.int32)
    return lax.shift_right_logical(
        bits + 0x7FFF + (lax.shift_right_logical(bits, 16) & 1), 16)


def _diff_body(a_ref, b_ref, o_ref, *, hh):
    a = a_ref[...]
    b = b_ref[...]
    plo = _pat16(a[:, :, :hh, :], b[:, :, :hh, :])
    phi = _pat16(a[:, :, hh:, :], b[:, :, hh:, :])
    o_ref[...] = plo | lax.shift_left(phi, 16)


def _diff(x, y):
    n, c, h, w = x.shape
    hh = h // 2
    bc = 8
    spec = pl.BlockSpec((1, bc, h, w), lambda i, j: (i, j, 0, 0))
    return pl.pallas_call(
        functools.partial(_diff_body, hh=hh),
        out_shape=jax.ShapeDtypeStruct((n, c, hh, w), jnp.int32),
        grid=(n, c // bc),
        in_specs=[spec, spec],
        out_specs=pl.BlockSpec((1, bc, hh, w), lambda i, j: (i, j, 0, 0)),
    )(x, y)


def _mean_body(*refs, denom):
    o_ref = refs[-1]
    total = refs[0][...]
    for r in refs[1:-1]:
        total = total + r[...]
    o_ref[...] = jnp.reshape(jnp.sum(total) * (1.0 / denom), (1, 1))


def _mean(ps, denom):
    return pl.pallas_call(
        functools.partial(_mean_body, denom=denom),
        out_shape=jax.ShapeDtypeStruct((1, 1), jnp.float32),
    )(*ps)


def _make_sc_topk(n, c, hh, w, k):
    rows = n * c
    rpw = rows // _NW
    ngrp = w // 16
    mesh = plsc.VectorSubcoreMesh(core_axis_name="c", subcore_axis_name="s")

    @functools.partial(
        pl.kernel,
        out_type=jax.ShapeDtypeStruct((_NW, 128), jnp.float32),
        mesh=mesh,
        compiler_params=pltpu.CompilerParams(needs_layout_passes=False),
        scratch_types=[
            pltpu.VMEM((hh // 2, w), jnp.int32),
            pltpu.VMEM((hh // 2, w), jnp.int32),
            pltpu.VMEM((16 * _NB,), jnp.int32),
            pltpu.VMEM((_NB,), jnp.int32),
            pltpu.VMEM((256,), jnp.int32),
            pltpu.VMEM((16,), jnp.float32),
            pltpu.VMEM((128,), jnp.float32),
            pltpu.SemaphoreType.DMA,
            pltpu.SemaphoreType.DMA,
        ],
    )
    def sc_topk(diff_hbm, out_hbm, buf_a, buf_b, hist, counts, hist2, pacc,
                obuf, sem_a, sem_b):
        cid = lax.axis_index("c")
        sid = lax.axis_index("s")
        wid = sid * 2 + cid
        lanes = lax.iota(jnp.int32, 16)
        ones = jnp.ones((16,), jnp.int32)
        izeros = jnp.zeros((16,), jnp.int32)
        fzeros = jnp.zeros((16,), jnp.float32)
        lane_hist = lanes * _NB
        lane_h2 = lanes * 16

        @plsc.parallel_loop(0, 16 * _NB // 16, unroll=4)
        def zero_hist(j):
            hist[pl.ds(j * 16, 16)] = izeros

        hh2 = hh // 2

        def nc_of(r):
            rn = r // c
            return rn, r - rn * c

        def dma(buf, sem, r, half):
            rn, rc = nc_of(r)
            return pltpu.make_async_copy(
                diff_hbm.at[rn, rc, pl.ds(half * hh2, hh2)], buf, sem)

        def p1_half(buf):
            @plsc.parallel_loop(0, hh2, unroll=4)
            def p1(s):
                for t in range(ngrp):
                    u = buf[s, pl.ds(t * 16, 16)]
                    blo = lax.shift_right_logical(u & 0xFFFF, 4)
                    bhi = lax.shift_right_logical(u, 20)
                    plsc.addupdate_scatter(hist, [lane_hist + blo], ones)
                    plsc.addupdate_scatter(hist, [lane_hist + bhi], ones)

        dma(buf_a, sem_a, wid * rpw, 0).start()

        def row_step(i, acc):
            r = wid * rpw + i
            dma(buf_a, sem_a, r, 0).wait()
            dma(buf_b, sem_b, r, 1).start()
            p1_half(buf_a)
            dma(buf_b, sem_b, r, 1).wait()
            p1_half(buf_b)

            @plsc.parallel_loop(0, _NBC, unroll=2)
            def lred(cc):
                b0 = cc * 16
                v = hist[pl.ds(b0, 16)]
                hist[pl.ds(b0, 16)] = izeros
                for l in range(1, 16):
                    o = l * _NB + b0
                    v = v + hist[pl.ds(o, 16)]
                    hist[pl.ds(o, 16)] = izeros
                counts[pl.ds(b0, 16)] = v

            def walk(cc, st):
                cum, ccf, found = st
                v = counts[pl.ds((_NBC - 1 - cc) * 16, 16)]
                tot = jnp.sum(v)
                crosses = jnp.logical_and(found == 0, cum + tot >= k)
                hold = jnp.logical_or(found == 1, crosses)
                return (jnp.where(hold, cum, cum + tot),
                        jnp.where(crosses, cc, ccf),
                        jnp.where(hold, 1, 0))
            cumf, ccf, _ = lax.fori_loop(
                0, _NBC, walk, (jnp.int32(0), jnp.int32(0), jnp.int32(0)))
            b0 = (_NBC - 1 - ccf) * 16
            v = counts[pl.ds(b0, 16)]
            rev = lax.rev(v, (0,))
            cs = plsc.cumsum(rev) + cumf
            reached = cs >= k
            p = jnp.min(jnp.where(reached, lanes, 16))
            cb_p = jnp.sum(jnp.where(lanes == p, cs - rev, 0))
            tb = b0 + 15 - p
            need = k - cb_p

            for l in range(16):
                hist2[pl.ds(l * 16, 16)] = izeros
            pacc[pl.ds(0, 16)] = fzeros

            base0 = lax.shift_left(tb, 20)
            lim = base0 + (1 << 20)

            def p2_half(buf):
                @plsc.parallel_loop(0, hh2, unroll=4)
                def p2(s):
                    for t in range(ngrp):
                        u = buf[s, pl.ds(t * 16, 16)]
                        t1 = lax.shift_left(u, 16)
                        t2 = u & jnp.int32(-65536)
                        for tv, low4 in (
                            (t1, u & 15),
                            (t2, lax.shift_right_logical(u, 16) & 15),
                        ):
                            gt = tv >= lim
                            plsc.addupdate_scatter(
                                pacc, [lanes],
                                plsc.bitcast(tv, jnp.float32), mask=gt)
                            plsc.addupdate_scatter(
                                hist2, [lane_h2 + low4], ones,
                                mask=(tv >= base0) ^ gt)
                return p2

            p2_half(buf_a)

            @pl.when(i + 1 < rpw)
            def _prefetch():
                dma(buf_a, sem_a, r + 1, 0).start()

            p2_half(buf_b)
            accv = pacc[pl.ds(0, 16)]

            c2 = hist2[pl.ds(0, 16)]
            for l in range(1, 16):
                c2 = c2 + hist2[pl.ds(l * 16, 16)]
            rev2 = lax.rev(c2, (0,))
            cs2 = plsc.cumsum(rev2)
            take = jnp.clip(need - (cs2 - rev2), 0, rev2)
            pat = lax.shift_left(tb, 4) + (15 - lanes)
            vals = plsc.bitcast(lax.shift_left(pat, 16), jnp.float32)
            return acc + accv + take.astype(jnp.float32) * vals

        acc = lax.fori_loop(0, rpw, row_step, fzeros)
        for l in range(8):
            obuf[pl.ds(l * 16, 16)] = fzeros
        obuf[pl.ds(0, 16)] = acc
        pltpu.sync_copy(obuf, out_hbm.at[wid])

    return sc_topk


def kernel(input, target):
    n, c, h, w = input.shape
    k = int(h * w * _RATIO)
    packed = _diff(input, target)
    partials = _make_sc_topk(n, c, h // 2, w, k)(packed)
    out = _mean([partials], float(n * c) * float(k))
    return out.reshape(())


# speedup vs baseline: 1.0938x; 1.0938x over previous
import functools

import jax
import jax.numpy as jnp
from jax import lax
from jax.experimental import pallas as pl
from jax.experimental.pallas import tpu as pltpu
from jax.experimental.pallas import tpu_sc as plsc

_RATIO = 0.1
_NW = 32
_NB = 2048
_NBC = _NB // 16


def _pat16(a, b):
    bits = jax.lax.bitcast_convert_type(jnp.abs(a - b), jnp.int32)
    return lax.shift_right_logical(
        bits + 0x7FFF + (lax.shift_right_logical(bits, 16) & 1), 16)


def _diff_body(a_ref, b_ref, o_ref, *, hh):
    a = a_ref[...]
    b = b_ref[...]
    plo = _pat16(a[:, :, :hh, :], b[:, :, :hh, :])
    phi = _pat16(a[:, :, hh:, :], b[:, :, hh:, :])
    o_ref[...] = plo | lax.shift_left(phi, 16)


def _diff(x, y):
    n, c, h, w = x.shape
    hh = h // 2
    bc = 8
    spec = pl.BlockSpec((1, bc, h, w), lambda i, j: (i, j, 0, 0))
    return pl.pallas_call(
        functools.partial(_diff_body, hh=hh),
        out_shape=jax.ShapeDtypeStruct((n, c, hh, w), jnp.int32),
        grid=(n, c // bc),
        in_specs=[spec, spec],
        out_specs=pl.BlockSpec((1, bc, hh, w), lambda i, j: (i, j, 0, 0)),
    )(x, y)


def _mean_body(*refs, denom):
    o_ref = refs[-1]
    total = refs[0][...]
    for r in refs[1:-1]:
        total = total + r[...]
    o_ref[...] = jnp.reshape(jnp.sum(total) * (1.0 / denom), (1, 1))


def _mean(ps, denom):
    return pl.pallas_call(
        functools.partial(_mean_body, denom=denom),
        out_shape=jax.ShapeDtypeStruct((1, 1), jnp.float32),
    )(*ps)


def _make_sc_topk(n, c, hh, w, k):
    rows = n * c
    rpw = rows // _NW
    ngrp = w // 16
    mesh = plsc.VectorSubcoreMesh(core_axis_name="c", subcore_axis_name="s")

    @functools.partial(
        pl.kernel,
        out_type=jax.ShapeDtypeStruct((_NW, 128), jnp.float32),
        mesh=mesh,
        compiler_params=pltpu.CompilerParams(needs_layout_passes=False),
        scratch_types=[
            pltpu.VMEM((hh // 2, w), jnp.int32),
            pltpu.VMEM((hh // 2, w), jnp.int32),
            pltpu.VMEM((16 * _NB,), jnp.int32),
            pltpu.VMEM((_NB,), jnp.int32),
            pltpu.VMEM((256,), jnp.int32),
            pltpu.VMEM((16,), jnp.float32),
            pltpu.VMEM((128,), jnp.float32),
            pltpu.SemaphoreType.DMA,
            pltpu.SemaphoreType.DMA,
        ],
    )
    def sc_topk(diff_hbm, out_hbm, buf_a, buf_b, hist, counts, hist2, pacc,
                obuf, sem_a, sem_b):
        cid = lax.axis_index("c")
        sid = lax.axis_index("s")
        wid = sid * 2 + cid
        lanes = lax.iota(jnp.int32, 16)
        ones = jnp.ones((16,), jnp.int32)
        izeros = jnp.zeros((16,), jnp.int32)
        fzeros = jnp.zeros((16,), jnp.float32)
        lane_hist = lanes * _NB
        lane_h2 = lanes * 16

        @plsc.parallel_loop(0, 16 * _NB // 16, unroll=4)
        def zero_hist(j):
            hist[pl.ds(j * 16, 16)] = izeros

        hh2 = hh // 2

        def nc_of(r):
            rn = r // c
            return rn, r - rn * c

        def dma(buf, sem, r, half):
            rn, rc = nc_of(r)
            return pltpu.make_async_copy(
                diff_hbm.at[rn, rc, pl.ds(half * hh2, hh2)], buf, sem)

        def p1_half(buf):
            @plsc.parallel_loop(0, hh2, unroll=2)
            def p1(s):
                for t in range(ngrp):
                    u = buf[s, pl.ds(t * 16, 16)]
                    blo = lax.shift_right_logical(u & 0xFFFF, 4)
                    bhi = lax.shift_right_logical(u, 20)
                    plsc.addupdate_scatter(hist, [lane_hist + blo], ones)
                    plsc.addupdate_scatter(hist, [lane_hist + bhi], ones)

        dma(buf_a, sem_a, wid * rpw, 0).start()

        def row_step(i, acc):
            r = wid * rpw + i
            dma(buf_a, sem_a, r, 0).wait()
            dma(buf_b, sem_b, r, 1).start()
            p1_half(buf_a)
            dma(buf_b, sem_b, r, 1).wait()
            p1_half(buf_b)

            @plsc.parallel_loop(0, _NBC, unroll=2)
            def lred(cc):
                b0 = cc * 16
                v = hist[pl.ds(b0, 16)]
                hist[pl.ds(b0, 16)] = izeros
                for l in range(1, 16):
                    o = l * _NB + b0
                    v = v + hist[pl.ds(o, 16)]
                    hist[pl.ds(o, 16)] = izeros
                counts[pl.ds(b0, 16)] = v

            def walk(cc, st):
                cum, ccf, found = st
                v = counts[pl.ds((_NBC - 1 - cc) * 16, 16)]
                tot = jnp.sum(v)
                crosses = jnp.logical_and(found == 0, cum + tot >= k)
                hold = jnp.logical_or(found == 1, crosses)
                return (jnp.where(hold, cum, cum + tot),
                        jnp.where(crosses, cc, ccf),
                        jnp.where(hold, 1, 0))
            cumf, ccf, _ = lax.fori_loop(
                0, _NBC, walk, (jnp.int32(0), jnp.int32(0), jnp.int32(0)))
            b0 = (_NBC - 1 - ccf) * 16
            v = counts[pl.ds(b0, 16)]
            rev = lax.rev(v, (0,))
            cs = plsc.cumsum(rev) + cumf
            reached = cs >= k
            p = jnp.min(jnp.where(reached, lanes, 16))
            cb_p = jnp.sum(jnp.where(lanes == p, cs - rev, 0))
            tb = b0 + 15 - p
            need = k - cb_p

            for l in range(16):
                hist2[pl.ds(l * 16, 16)] = izeros
            pacc[pl.ds(0, 16)] = fzeros

            base0 = lax.shift_left(tb, 20)
            lim = base0 + (1 << 20)

            def p2_half(buf):
                @plsc.parallel_loop(0, hh2, unroll=2)
                def p2(s):
                    for t in range(ngrp):
                        u = buf[s, pl.ds(t * 16, 16)]
                        t1 = lax.shift_left(u, 16)
                        t2 = u & jnp.int32(-65536)
                        for tv, low4 in (
                            (t1, u & 15),
                            (t2, lax.shift_right_logical(u, 16) & 15),
                        ):
                            gt = tv >= lim
                            plsc.addupdate_scatter(
                                pacc, [lanes],
                                plsc.bitcast(tv, jnp.float32), mask=gt)
                            plsc.addupdate_scatter(
                                hist2, [lane_h2 + low4], ones,
                                mask=(tv >= base0) ^ gt)
                return p2

            p2_half(buf_a)

            @pl.when(i + 1 < rpw)
            def _prefetch():
                dma(buf_a, sem_a, r + 1, 0).start()

            p2_half(buf_b)
            accv = pacc[pl.ds(0, 16)]

            c2 = hist2[pl.ds(0, 16)]
            for l in range(1, 16):
                c2 = c2 + hist2[pl.ds(l * 16, 16)]
            rev2 = lax.rev(c2, (0,))
            cs2 = plsc.cumsum(rev2)
            take = jnp.clip(need - (cs2 - rev2), 0, rev2)
            pat = lax.shift_left(tb, 4) + (15 - lanes)
            vals = plsc.bitcast(lax.shift_left(pat, 16), jnp.float32)
            return acc + accv + take.astype(jnp.float32) * vals

        acc = lax.fori_loop(0, rpw, row_step, fzeros)
        for l in range(8):
            obuf[pl.ds(l * 16, 16)] = fzeros
        obuf[pl.ds(0, 16)] = acc
        pltpu.sync_copy(obuf, out_hbm.at[wid])

    return sc_topk


def kernel(input, target):
    n, c, h, w = input.shape
    k = int(h * w * _RATIO)
    packed = _diff(input, target)
    partials = _make_sc_topk(n, c, h // 2, w, k)(packed)
    out = _mean([partials], float(n * c) * float(k))
    return out.reshape(())
